# Initial kernel scaffold; baseline (speedup 1.0000x reference)
import jax
import jax.numpy as jnp
from jax.experimental import pallas as pl


def kernel(x, adj_t, Wl1, Wr1, att1, b1, Wl2, Wr2, att2, b2):
    raise NotImplementedError("write your pallas kernel here")


# speedup vs baseline: 1.0001x; 1.0001x over previous
import jax
import jax.numpy as jnp
from jax.experimental import pallas as pl

_N = 10000


def _layer(x, src, dst, Wl, Wr, att, b, heads, concat):
    n = x.shape[0]
    out_ch = att.shape[1]
    xl = (x @ Wl).reshape(n, heads, out_ch)
    xr = (x @ Wr).reshape(n, heads, out_ch)
    xj = jnp.take(xl, src, axis=0)
    xi = jnp.take(xr, dst, axis=0)
    e = jax.nn.leaky_relu(xj + xi, negative_slope=0.2)
    alpha = jnp.sum(e * att[None, :, :], axis=-1)
    amax = jax.ops.segment_max(alpha, dst, num_segments=n)
    alpha = jnp.exp(alpha - jnp.take(amax, dst, axis=0))
    denom = jax.ops.segment_sum(alpha, dst, num_segments=n)
    alpha = alpha / (jnp.take(denom, dst, axis=0) + 1e-16)
    out = jax.ops.segment_sum(xj * alpha[:, :, None], dst, num_segments=n)
    if concat:
        out = out.reshape(n, heads * out_ch)
    else:
        out = out.mean(axis=1)
    return out + b


def kernel(x, adj_t, Wl1, Wr1, att1, b1, Wl2, Wr2, att2, b2):
    loops = jnp.arange(_N, dtype=adj_t.dtype)
    src = jnp.concatenate([adj_t[0], loops])
    dst = jnp.concatenate([adj_t[1], loops])
    h = _layer(x, src, dst, Wl1, Wr1, att1, b1, 8, True)
    h = jax.nn.relu(h)
    out = _layer(h, src, dst, Wl2, Wr2, att2, b2, 1, False)
    return out


# speedup vs baseline: 19.0488x; 19.0471x over previous
import functools

import jax
import jax.numpy as jnp
from jax import lax
from jax.experimental import pallas as pl
from jax.experimental.pallas import tpu as pltpu
from jax.experimental.pallas import tpu_sc as plsc

_N = 10000
_E = 320000
_NP = 10240
_EP = 331776
_ROWS_PER_TILE = _NP // 16

_mesh = plsc.VectorSubcoreMesh(core_axis_name="c", subcore_axis_name="s")


def _sc_layer(D, H, ROWLEN, K):
    CH = D // H
    BLK_PER_TILE = _EP // (32 * K)

    @functools.partial(
        pl.kernel,
        mesh=_mesh,
        compiler_params=pltpu.CompilerParams(
            needs_layout_passes=False, use_tc_tiling_on_sc=False),
        out_type=jax.ShapeDtypeStruct((2, _NP, ROWLEN), jnp.float32),
        scratch_types=[
            pltpu.VMEM_SHARED((_NP, ROWLEN), jnp.float32),
            pltpu.VMEM((K,), jnp.int32),
            pltpu.VMEM((K,), jnp.int32),
            pltpu.VMEM((K, D), jnp.float32),
            pltpu.VMEM((K, D), jnp.float32),
            pltpu.VMEM((K, ROWLEN), jnp.float32),
            pltpu.VMEM((D,), jnp.float32),
            pltpu.SemaphoreType.DMA,
            pltpu.SemaphoreType.DMA,
        ],
    )
    def body(xl_h, xr_h, src_h, dst_h, att_h, out_h,
             acc_s, idx_s, idx_d, xlb, xrb, wxlb, att_v, sem1, sem2):
        c = lax.axis_index("c")
        s = lax.axis_index("s")
        wid = s * 2 + c

        pltpu.sync_copy(att_h, att_v)

        zv = jnp.zeros((16,), jnp.float32)

        def zb(r, carry):
            for cc in range(ROWLEN // 16):
                wxlb[r, pl.ds(cc * 16, 16)] = zv
            return carry

        lax.fori_loop(0, K, zb, 0)
        for t in range(_ROWS_PER_TILE // K):
            pltpu.sync_copy(wxlb, acc_s.at[pl.ds(s * _ROWS_PER_TILE + t * K, K)])
        plsc.subcore_barrier()

        def blk(j, carry):
            base = (wid * BLK_PER_TILE + j) * K
            pltpu.sync_copy(src_h.at[pl.ds(base, K)], idx_s)
            pltpu.sync_copy(dst_h.at[pl.ds(base, K)], idx_d)
            cp1 = pltpu.async_copy(xl_h.at[idx_s], xlb, sem1)
            cp2 = pltpu.async_copy(xr_h.at[idx_d], xrb, sem2)
            cp1.wait()
            cp2.wait()

            def eg_body(eg, carry2):
                rowv = eg * 16 + lax.iota(jnp.int32, 16)
                for h in range(H):
                    acc = jnp.zeros((16,), jnp.float32)
                    att_rows = [att_v[pl.ds(h * CH + k * 16, 16)]
                                for k in range(CH // 16)]
                    saved = []
                    for c0 in range(CH):
                        ch = h * CH + c0
                        colv = jnp.full((16,), ch, jnp.int32)
                        xlv = plsc.load_gather(xlb, [rowv, colv])
                        xrv = plsc.load_gather(xrb, [rowv, colv])
                        sv = xlv + xrv
                        lv = jnp.maximum(sv, sv * 0.2)
                        acc = acc + lv * att_rows[c0 // 16][c0 % 16]
                        if CH <= 16:
                            saved.append((colv, xlv))
                    wv = jnp.exp(acc)
                    if CH <= 16:
                        for colv, xlv in saved:
                            plsc.store_scatter(wxlb, [rowv, colv], xlv * wv)
                    else:
                        for c0 in range(CH):
                            ch = h * CH + c0
                            colv = jnp.full((16,), ch, jnp.int32)
                            xlv = plsc.load_gather(xlb, [rowv, colv])
                            plsc.store_scatter(wxlb, [rowv, colv], xlv * wv)
                    plsc.store_scatter(
                        wxlb, [rowv, jnp.full((16,), D + h, jnp.int32)], wv)
                return carry2

            lax.fori_loop(0, K // 16, eg_body, 0)
            pltpu.sync_copy(wxlb, acc_s.at[idx_d], add=True)
            return carry

        lax.fori_loop(0, BLK_PER_TILE, blk, 0)
        plsc.subcore_barrier()
        pltpu.sync_copy(acc_s.at[pl.ds(s * _ROWS_PER_TILE, _ROWS_PER_TILE)],
                        out_h.at[c, pl.ds(s * _ROWS_PER_TILE, _ROWS_PER_TILE)])

    return body


_sc_layer1 = _sc_layer(128, 8, 144, 64)
_sc_layer2 = _sc_layer(48, 1, 64, 128)


def _tc_mm2(x, Wa, Wb):
    n, f = x.shape
    d = Wa.shape[1]
    B = 1024

    def body(x_r, wa_r, wb_r, oa_r, ob_r):
        xb = x_r[...]
        oa_r[...] = jnp.dot(xb, wa_r[...], preferred_element_type=jnp.float32)
        ob_r[...] = jnp.dot(xb, wb_r[...], preferred_element_type=jnp.float32)

    return pl.pallas_call(
        body,
        grid=(n // B,),
        in_specs=[
            pl.BlockSpec((B, f), lambda i: (i, 0)),
            pl.BlockSpec((f, d), lambda i: (0, 0)),
            pl.BlockSpec((f, d), lambda i: (0, 0)),
        ],
        out_specs=[
            pl.BlockSpec((B, d), lambda i: (i, 0)),
            pl.BlockSpec((B, d), lambda i: (i, 0)),
        ],
        out_shape=[
            jax.ShapeDtypeStruct((n, d), jnp.float32),
            jax.ShapeDtypeStruct((n, d), jnp.float32),
        ],
    )(x, Wa, Wb)


def _tc_combine1_mm(p0, p1, b1, Wl2, Wr2):
    n = p0.shape[0]
    B = 1024
    d2 = Wl2.shape[1]

    def body(p0_r, p1_r, b1_r, wl_r, wr_r, oa_r, ob_r):
        p = p0_r[...] + p1_r[...]
        num = p[:, :128]
        den = p[:, 128:136]
        denb = jnp.broadcast_to(den.reshape(B, 8, 1), (B, 8, 16)).reshape(B, 128)
        h = jnp.maximum(num / (denb + 1e-16) + b1_r[...], 0.0)
        oa_r[...] = jnp.dot(h, wl_r[...], preferred_element_type=jnp.float32)
        ob_r[...] = jnp.dot(h, wr_r[...], preferred_element_type=jnp.float32)

    return pl.pallas_call(
        body,
        grid=(n // B,),
        in_specs=[
            pl.BlockSpec((B, 144), lambda i: (i, 0)),
            pl.BlockSpec((B, 144), lambda i: (i, 0)),
            pl.BlockSpec((1, 128), lambda i: (0, 0)),
            pl.BlockSpec((128, d2), lambda i: (0, 0)),
            pl.BlockSpec((128, d2), lambda i: (0, 0)),
        ],
        out_specs=[
            pl.BlockSpec((B, d2), lambda i: (i, 0)),
            pl.BlockSpec((B, d2), lambda i: (i, 0)),
        ],
        out_shape=[
            jax.ShapeDtypeStruct((n, d2), jnp.float32),
            jax.ShapeDtypeStruct((n, d2), jnp.float32),
        ],
    )(p0, p1, b1, Wl2, Wr2)


def _tc_combine2(q0, q1, b2):
    n = q0.shape[0]
    B = 1024

    def body(q0_r, q1_r, b2_r, o_r):
        q = q0_r[...] + q1_r[...]
        den = jnp.broadcast_to(q[:, 48:49], (B, 64))
        o_r[...] = q / (den + 1e-16) + b2_r[...]

    return pl.pallas_call(
        body,
        grid=(n // B,),
        in_specs=[
            pl.BlockSpec((B, 64), lambda i: (i, 0)),
            pl.BlockSpec((B, 64), lambda i: (i, 0)),
            pl.BlockSpec((1, 64), lambda i: (0, 0)),
        ],
        out_specs=pl.BlockSpec((B, 64), lambda i: (i, 0)),
        out_shape=jax.ShapeDtypeStruct((n, 64), jnp.float32),
    )(q0, q1, b2)


def kernel(x, adj_t, Wl1, Wr1, att1, b1, Wl2, Wr2, att2, b2):
    loops = jnp.arange(_N, dtype=jnp.int32)
    padi = jnp.full((_EP - _E - _N,), _N, dtype=jnp.int32)
    src = jnp.concatenate([adj_t[0].astype(jnp.int32), loops, padi])
    dst = jnp.concatenate([adj_t[1].astype(jnp.int32), loops, padi])

    xp = jnp.pad(x, ((0, _NP - _N), (0, 0)))
    xl1, xr1 = _tc_mm2(xp, Wl1, Wr1)

    att1f = att1.reshape(128)
    parts1 = _sc_layer1(xl1, xr1, src, dst, att1f)

    Wl2p = jnp.pad(Wl2, ((0, 0), (0, 8)))
    Wr2p = jnp.pad(Wr2, ((0, 0), (0, 8)))
    hl2, hr2 = _tc_combine1_mm(parts1[0], parts1[1], b1.reshape(1, 128),
                               Wl2p, Wr2p)

    att2f = jnp.pad(att2.reshape(40), (0, 8))
    parts2 = _sc_layer2(hl2, hr2, src, dst, att2f)

    b2p = jnp.pad(b2, (0, 24)).reshape(1, 64)
    outp = _tc_combine2(parts2[0], parts2[1], b2p)
    return outp[:_N, :40]


# speedup vs baseline: 23.5008x; 1.2337x over previous
import functools

import jax
import jax.numpy as jnp
from jax import lax
from jax.experimental import pallas as pl
from jax.experimental.pallas import tpu as pltpu
from jax.experimental.pallas import tpu_sc as plsc

_N = 10000
_E = 320000
_NP = 10240
_EP = 331776
_ROWS_PER_TILE = _NP // 16

_mesh = plsc.VectorSubcoreMesh(core_axis_name="c", subcore_axis_name="s")


def _sc_layer(D, H, ROWLEN, K):
    CH = D // H
    NB = _EP // (32 * K)
    CHUNK_G = 27
    CHUNK_B = CHUNK_G * 2
    NCHUNK = NB // CHUNK_B

    @functools.partial(
        pl.kernel,
        mesh=_mesh,
        compiler_params=pltpu.CompilerParams(
            needs_layout_passes=False, use_tc_tiling_on_sc=False),
        out_type=jax.ShapeDtypeStruct((2, _NP, ROWLEN), jnp.float32),
        scratch_types=[
            pltpu.VMEM_SHARED((_NP, ROWLEN), jnp.float32),
            pltpu.VMEM((CHUNK_B, 2 * K), jnp.int32),
            pltpu.VMEM((CHUNK_B, K), jnp.int32),
            pltpu.VMEM((2, 2 * K, D), jnp.float32),
            pltpu.VMEM((2, K, ROWLEN), jnp.float32),
            pltpu.VMEM((D,), jnp.float32),
            pltpu.SemaphoreType.DMA,
            pltpu.SemaphoreType.DMA,
            pltpu.SemaphoreType.DMA,
            pltpu.SemaphoreType.DMA,
        ],
    )
    def body(t_h, gidx_h, dst2_h, att_h, out_h,
             acc_s, gidxb, dstb, xlrb, wxlb, att_v, g0, g1, s0, s1):
        c = lax.axis_index("c")
        s = lax.axis_index("s")
        wid = s * 2 + c
        wbase = wid * NB

        pltpu.sync_copy(att_h, att_v)

        zv = jnp.zeros((16,), jnp.float32)

        def zb(r, carry):
            for b in range(2):
                for cc in range(ROWLEN // 16):
                    wxlb[b, r, pl.ds(cc * 16, 16)] = zv
            return carry

        lax.fori_loop(0, K, zb, 0)
        for t in range(_ROWS_PER_TILE // K):
            pltpu.sync_copy(wxlb.at[0],
                            acc_s.at[pl.ds(s * _ROWS_PER_TILE + t * K, K)])
        plsc.subcore_barrier()

        gsems = (g0, g1)
        ssems = (s0, s1)

        def drain_scatter(b):
            pltpu.make_async_copy(out_h.at[0, pl.ds(0, K)], wxlb.at[b],
                                  ssems[b]).wait()

        def load_chunk(cc):
            pltpu.sync_copy(gidx_h.at[pl.ds(wbase + cc * CHUNK_B, CHUNK_B)],
                            gidxb)
            pltpu.sync_copy(dst2_h.at[pl.ds(wbase + cc * CHUNK_B, CHUNK_B)],
                            dstb)

        def issue_gather(jl, b):
            pltpu.async_copy(t_h.at[gidxb.at[jl]], xlrb.at[b], gsems[b])

        load_chunk(0)
        issue_gather(0, 0)
        issue_gather(1, 1)

        def chunk(cc, carry):
            @pl.when(cc > 0)
            def _():
                drain_scatter(0)
                drain_scatter(1)
                load_chunk(cc)
                issue_gather(0, 0)
                issue_gather(1, 1)

            def group(jj, carry2):
                for b in range(2):
                    jl = jj * 2 + b
                    pltpu.make_async_copy(t_h.at[gidxb.at[jl]], xlrb.at[b],
                                          gsems[b]).wait()
                    @pl.when(jl >= 2)
                    def _():
                        drain_scatter(b)

                    def eg_body(eg, carry3):
                        rowv = eg * 16 + lax.iota(jnp.int32, 16)
                        xb = xlrb.at[b]
                        for h in range(H):
                            acc = jnp.zeros((16,), jnp.float32)
                            att_rows = [att_v[pl.ds(h * CH + k * 16, 16)]
                                        for k in range(CH // 16)]
                            saved = []
                            for c0 in range(CH):
                                ch = h * CH + c0
                                colv = jnp.full((16,), ch, jnp.int32)
                                xlv = plsc.load_gather(xb, [rowv, colv])
                                xrv = plsc.load_gather(xb, [rowv + K, colv])
                                sv = xlv + xrv
                                lv = jnp.maximum(sv, sv * 0.2)
                                acc = acc + lv * att_rows[c0 // 16][c0 % 16]
                                if CH <= 16:
                                    saved.append((colv, xlv))
                            wv = jnp.exp(acc)
                            if CH <= 16:
                                for colv, xlv in saved:
                                    plsc.store_scatter(wxlb.at[b],
                                                       [rowv, colv], xlv * wv)
                            else:
                                for c0 in range(CH):
                                    ch = h * CH + c0
                                    colv = jnp.full((16,), ch, jnp.int32)
                                    xlv = plsc.load_gather(xb, [rowv, colv])
                                    plsc.store_scatter(wxlb.at[b],
                                                       [rowv, colv], xlv * wv)
                            plsc.store_scatter(
                                wxlb.at[b],
                                [rowv, jnp.full((16,), D + h, jnp.int32)], wv)
                        return carry3

                    lax.fori_loop(0, K // 16, eg_body, 0)
                    pltpu.async_copy(wxlb.at[b], acc_s.at[dstb.at[jl]],
                                     ssems[b], add=True)
                    @pl.when(jl + 2 < CHUNK_B)
                    def _():
                        issue_gather(jl + 2, b)
                return carry2

            lax.fori_loop(0, CHUNK_G, group, 0)
            return carry

        lax.fori_loop(0, NCHUNK, chunk, 0)
        drain_scatter(0)
        drain_scatter(1)
        plsc.subcore_barrier()
        pltpu.sync_copy(acc_s.at[pl.ds(s * _ROWS_PER_TILE, _ROWS_PER_TILE)],
                        out_h.at[c, pl.ds(s * _ROWS_PER_TILE, _ROWS_PER_TILE)])

    return body


_K1 = 32
_K2 = 64
_sc_layer1 = _sc_layer(128, 8, 144, _K1)
_sc_layer2 = _sc_layer(48, 1, 64, _K2)


def _tc_mm2(x, Wa, Wb):
    n, f = x.shape
    d = Wa.shape[1]
    B = 1024

    def body(x_r, wa_r, wb_r, oa_r, ob_r):
        xb = x_r[...]
        oa_r[...] = jnp.dot(xb, wa_r[...], preferred_element_type=jnp.float32)
        ob_r[...] = jnp.dot(xb, wb_r[...], preferred_element_type=jnp.float32)

    return pl.pallas_call(
        body,
        grid=(n // B,),
        in_specs=[
            pl.BlockSpec((B, f), lambda i: (i, 0)),
            pl.BlockSpec((f, d), lambda i: (0, 0)),
            pl.BlockSpec((f, d), lambda i: (0, 0)),
        ],
        out_specs=[
            pl.BlockSpec((B, d), lambda i: (i, 0)),
            pl.BlockSpec((B, d), lambda i: (i, 0)),
        ],
        out_shape=[
            jax.ShapeDtypeStruct((n, d), jnp.float32),
            jax.ShapeDtypeStruct((n, d), jnp.float32),
        ],
    )(x, Wa, Wb)


def _tc_combine1_mm(p0, p1, b1, Wl2, Wr2):
    n = p0.shape[0]
    B = 1024
    d2 = Wl2.shape[1]

    def body(p0_r, p1_r, b1_r, wl_r, wr_r, oa_r, ob_r):
        p = p0_r[...] + p1_r[...]
        num = p[:, :128]
        den = p[:, 128:136]
        denb = jnp.broadcast_to(den.reshape(B, 8, 1), (B, 8, 16)).reshape(B, 128)
        h = jnp.maximum(num / (denb + 1e-16) + b1_r[...], 0.0)
        oa_r[...] = jnp.dot(h, wl_r[...], preferred_element_type=jnp.float32)
        ob_r[...] = jnp.dot(h, wr_r[...], preferred_element_type=jnp.float32)

    return pl.pallas_call(
        body,
        grid=(n // B,),
        in_specs=[
            pl.BlockSpec((B, 144), lambda i: (i, 0)),
            pl.BlockSpec((B, 144), lambda i: (i, 0)),
            pl.BlockSpec((1, 128), lambda i: (0, 0)),
            pl.BlockSpec((128, d2), lambda i: (0, 0)),
            pl.BlockSpec((128, d2), lambda i: (0, 0)),
        ],
        out_specs=[
            pl.BlockSpec((B, d2), lambda i: (i, 0)),
            pl.BlockSpec((B, d2), lambda i: (i, 0)),
        ],
        out_shape=[
            jax.ShapeDtypeStruct((n, d2), jnp.float32),
            jax.ShapeDtypeStruct((n, d2), jnp.float32),
        ],
    )(p0, p1, b1, Wl2, Wr2)


def _tc_combine2(q0, q1, b2):
    n = q0.shape[0]
    B = 1024

    def body(q0_r, q1_r, b2_r, o_r):
        q = q0_r[...] + q1_r[...]
        den = jnp.broadcast_to(q[:, 48:49], (B, 64))
        o_r[...] = q / (den + 1e-16) + b2_r[...]

    return pl.pallas_call(
        body,
        grid=(n // B,),
        in_specs=[
            pl.BlockSpec((B, 64), lambda i: (i, 0)),
            pl.BlockSpec((B, 64), lambda i: (i, 0)),
            pl.BlockSpec((1, 64), lambda i: (0, 0)),
        ],
        out_specs=pl.BlockSpec((B, 64), lambda i: (i, 0)),
        out_shape=jax.ShapeDtypeStruct((n, 64), jnp.float32),
    )(q0, q1, b2)


def kernel(x, adj_t, Wl1, Wr1, att1, b1, Wl2, Wr2, att2, b2):
    loops = jnp.arange(_N, dtype=jnp.int32)
    padi = jnp.full((_EP - _E - _N,), _N, dtype=jnp.int32)
    src = jnp.concatenate([adj_t[0].astype(jnp.int32), loops, padi])
    dst = jnp.concatenate([adj_t[1].astype(jnp.int32), loops, padi])

    gidx1 = jnp.concatenate(
        [src.reshape(-1, _K1), dst.reshape(-1, _K1) + _NP], axis=1)
    dstb1 = dst.reshape(-1, _K1)
    gidx2 = jnp.concatenate(
        [src.reshape(-1, _K2), dst.reshape(-1, _K2) + _NP], axis=1)
    dstb2 = dst.reshape(-1, _K2)

    xp = jnp.pad(x, ((0, _NP - _N), (0, 0)))
    xl1, xr1 = _tc_mm2(xp, Wl1, Wr1)
    t1 = jnp.concatenate([xl1, xr1])

    att1f = att1.reshape(128)
    parts1 = _sc_layer1(t1, gidx1, dstb1, att1f)

    Wl2p = jnp.pad(Wl2, ((0, 0), (0, 8)))
    Wr2p = jnp.pad(Wr2, ((0, 0), (0, 8)))
    hl2, hr2 = _tc_combine1_mm(parts1[0], parts1[1], b1.reshape(1, 128),
                               Wl2p, Wr2p)
    t2 = jnp.concatenate([hl2, hr2])

    att2f = jnp.pad(att2.reshape(40), (0, 8))
    parts2 = _sc_layer2(t2, gidx2, dstb2, att2f)

    b2p = jnp.pad(b2, (0, 24)).reshape(1, 64)
    outp = _tc_combine2(parts2[0], parts2[1], b2p)
    return outp[:_N, :40]


# speedup vs baseline: 23.5316x; 1.0013x over previous
import functools

import jax
import jax.numpy as jnp
from jax import lax
from jax.experimental import pallas as pl
from jax.experimental.pallas import tpu as pltpu
from jax.experimental.pallas import tpu_sc as plsc

_N = 10000
_E = 320000
_NP = 10240
_EP = 331776
_ROWS_PER_TILE = _NP // 16

_mesh = plsc.VectorSubcoreMesh(core_axis_name="c", subcore_axis_name="s")


def _sc_layer(D, H, ROWLEN, K):
    CH = D // H
    NB = _EP // (32 * K)
    CHUNK_G = 27
    CHUNK_B = CHUNK_G * 2
    NCHUNK = NB // CHUNK_B

    @functools.partial(
        pl.kernel,
        mesh=_mesh,
        compiler_params=pltpu.CompilerParams(
            needs_layout_passes=False, use_tc_tiling_on_sc=False),
        out_type=jax.ShapeDtypeStruct((2, _NP, ROWLEN), jnp.float32),
        scratch_types=[
            pltpu.VMEM_SHARED((_NP, ROWLEN), jnp.float32),
            pltpu.VMEM((CHUNK_B, 2 * K), jnp.int32),
            pltpu.VMEM((CHUNK_B, K), jnp.int32),
            pltpu.VMEM((2, 2 * K, D), jnp.float32),
            pltpu.VMEM((2, K, ROWLEN), jnp.float32),
            pltpu.VMEM((D,), jnp.float32),
            pltpu.SemaphoreType.DMA,
            pltpu.SemaphoreType.DMA,
            pltpu.SemaphoreType.DMA,
            pltpu.SemaphoreType.DMA,
            pltpu.SemaphoreType.DMA,
            pltpu.SemaphoreType.DMA,
        ],
    )
    def body(t_h, gidx_h, dst2_h, att_h, out_h,
             acc_s, gidxb, dstb, xlrb, wxlb, att_v, g0, g1, g2, g3, s0, s1):
        c = lax.axis_index("c")
        s = lax.axis_index("s")
        wid = s * 2 + c
        wbase = wid * NB

        pltpu.sync_copy(att_h, att_v)

        zv = jnp.zeros((16,), jnp.float32)

        def zb(r, carry):
            for b in range(2):
                for cc in range(ROWLEN // 16):
                    wxlb[b, r, pl.ds(cc * 16, 16)] = zv
            return carry

        lax.fori_loop(0, K, zb, 0)
        for t in range(_ROWS_PER_TILE // K):
            pltpu.sync_copy(wxlb.at[0],
                            acc_s.at[pl.ds(s * _ROWS_PER_TILE + t * K, K)])
        plsc.subcore_barrier()

        gsems = ((g0, g1), (g2, g3))
        ssems = (s0, s1)

        def drain_scatter(b):
            pltpu.make_async_copy(out_h.at[0, pl.ds(0, K)], wxlb.at[b],
                                  ssems[b]).wait()

        def load_chunk(cc):
            pltpu.sync_copy(gidx_h.at[pl.ds(wbase + cc * CHUNK_B, CHUNK_B)],
                            gidxb)
            pltpu.sync_copy(dst2_h.at[pl.ds(wbase + cc * CHUNK_B, CHUNK_B)],
                            dstb)

        def issue_gather(jl, b):
            pltpu.async_copy(t_h.at[gidxb.at[jl, pl.ds(0, K)]],
                             xlrb.at[b, pl.ds(0, K)], gsems[b][0])
            pltpu.async_copy(t_h.at[gidxb.at[jl, pl.ds(K, K)]],
                             xlrb.at[b, pl.ds(K, K)], gsems[b][1])

        def wait_gather(jl, b):
            pltpu.make_async_copy(t_h.at[gidxb.at[jl, pl.ds(0, K)]],
                                  xlrb.at[b, pl.ds(0, K)], gsems[b][0]).wait()
            pltpu.make_async_copy(t_h.at[gidxb.at[jl, pl.ds(K, K)]],
                                  xlrb.at[b, pl.ds(K, K)], gsems[b][1]).wait()

        load_chunk(0)
        issue_gather(0, 0)
        issue_gather(1, 1)

        def chunk(cc, carry):
            @pl.when(cc > 0)
            def _():
                drain_scatter(0)
                drain_scatter(1)
                load_chunk(cc)
                issue_gather(0, 0)
                issue_gather(1, 1)

            def group(jj, carry2):
                for b in range(2):
                    jl = jj * 2 + b
                    wait_gather(jl, b)
                    @pl.when(jl >= 2)
                    def _():
                        drain_scatter(b)

                    def eg_body(eg, carry3):
                        rowv = eg * 16 + lax.iota(jnp.int32, 16)
                        xb = xlrb.at[b]
                        for h in range(H):
                            acc = jnp.zeros((16,), jnp.float32)
                            att_rows = [att_v[pl.ds(h * CH + k * 16, 16)]
                                        for k in range(CH // 16)]
                            saved = []
                            for c0 in range(CH):
                                ch = h * CH + c0
                                colv = jnp.full((16,), ch, jnp.int32)
                                xlv = plsc.load_gather(xb, [rowv, colv])
                                xrv = plsc.load_gather(xb, [rowv + K, colv])
                                sv = xlv + xrv
                                lv = jnp.maximum(sv, sv * 0.2)
                                acc = acc + lv * att_rows[c0 // 16][c0 % 16]
                                if CH <= 16:
                                    saved.append((colv, xlv))
                            wv = jnp.exp(acc)
                            if CH <= 16:
                                for colv, xlv in saved:
                                    plsc.store_scatter(wxlb.at[b],
                                                       [rowv, colv], xlv * wv)
                            else:
                                for c0 in range(CH):
                                    ch = h * CH + c0
                                    colv = jnp.full((16,), ch, jnp.int32)
                                    xlv = plsc.load_gather(xb, [rowv, colv])
                                    plsc.store_scatter(wxlb.at[b],
                                                       [rowv, colv], xlv * wv)
                            plsc.store_scatter(
                                wxlb.at[b],
                                [rowv, jnp.full((16,), D + h, jnp.int32)], wv)
                        return carry3

                    lax.fori_loop(0, K // 16, eg_body, 0)
                    pltpu.async_copy(wxlb.at[b], acc_s.at[dstb.at[jl]],
                                     ssems[b], add=True)
                    @pl.when(jl + 2 < CHUNK_B)
                    def _():
                        issue_gather(jl + 2, b)
                return carry2

            lax.fori_loop(0, CHUNK_G, group, 0)
            return carry

        lax.fori_loop(0, NCHUNK, chunk, 0)
        drain_scatter(0)
        drain_scatter(1)
        plsc.subcore_barrier()
        pltpu.sync_copy(acc_s.at[pl.ds(s * _ROWS_PER_TILE, _ROWS_PER_TILE)],
                        out_h.at[c, pl.ds(s * _ROWS_PER_TILE, _ROWS_PER_TILE)])

    return body


_K1 = 32
_K2 = 96
_sc_layer1 = _sc_layer(128, 8, 144, _K1)
_sc_layer2 = _sc_layer(48, 1, 64, _K2)


def _tc_mm2(x, Wa, Wb):
    n, f = x.shape
    d = Wa.shape[1]
    B = 1024

    def body(x_r, wa_r, wb_r, oa_r, ob_r):
        xb = x_r[...]
        oa_r[...] = jnp.dot(xb, wa_r[...], preferred_element_type=jnp.float32)
        ob_r[...] = jnp.dot(xb, wb_r[...], preferred_element_type=jnp.float32)

    return pl.pallas_call(
        body,
        grid=(n // B,),
        in_specs=[
            pl.BlockSpec((B, f), lambda i: (i, 0)),
            pl.BlockSpec((f, d), lambda i: (0, 0)),
            pl.BlockSpec((f, d), lambda i: (0, 0)),
        ],
        out_specs=[
            pl.BlockSpec((B, d), lambda i: (i, 0)),
            pl.BlockSpec((B, d), lambda i: (i, 0)),
        ],
        out_shape=[
            jax.ShapeDtypeStruct((n, d), jnp.float32),
            jax.ShapeDtypeStruct((n, d), jnp.float32),
        ],
    )(x, Wa, Wb)


def _tc_combine1_mm(p0, p1, b1, Wl2, Wr2):
    n = p0.shape[0]
    B = 1024
    d2 = Wl2.shape[1]

    def body(p0_r, p1_r, b1_r, wl_r, wr_r, oa_r, ob_r):
        p = p0_r[...] + p1_r[...]
        num = p[:, :128]
        den = p[:, 128:136]
        denb = jnp.broadcast_to(den.reshape(B, 8, 1), (B, 8, 16)).reshape(B, 128)
        h = jnp.maximum(num / (denb + 1e-16) + b1_r[...], 0.0)
        oa_r[...] = jnp.dot(h, wl_r[...], preferred_element_type=jnp.float32)
        ob_r[...] = jnp.dot(h, wr_r[...], preferred_element_type=jnp.float32)

    return pl.pallas_call(
        body,
        grid=(n // B,),
        in_specs=[
            pl.BlockSpec((B, 144), lambda i: (i, 0)),
            pl.BlockSpec((B, 144), lambda i: (i, 0)),
            pl.BlockSpec((1, 128), lambda i: (0, 0)),
            pl.BlockSpec((128, d2), lambda i: (0, 0)),
            pl.BlockSpec((128, d2), lambda i: (0, 0)),
        ],
        out_specs=[
            pl.BlockSpec((B, d2), lambda i: (i, 0)),
            pl.BlockSpec((B, d2), lambda i: (i, 0)),
        ],
        out_shape=[
            jax.ShapeDtypeStruct((n, d2), jnp.float32),
            jax.ShapeDtypeStruct((n, d2), jnp.float32),
        ],
    )(p0, p1, b1, Wl2, Wr2)


def _tc_combine2(q0, q1, b2):
    n = q0.shape[0]
    B = 1024

    def body(q0_r, q1_r, b2_r, o_r):
        q = q0_r[...] + q1_r[...]
        den = jnp.broadcast_to(q[:, 48:49], (B, 64))
        o_r[...] = q / (den + 1e-16) + b2_r[...]

    return pl.pallas_call(
        body,
        grid=(n // B,),
        in_specs=[
            pl.BlockSpec((B, 64), lambda i: (i, 0)),
            pl.BlockSpec((B, 64), lambda i: (i, 0)),
            pl.BlockSpec((1, 64), lambda i: (0, 0)),
        ],
        out_specs=pl.BlockSpec((B, 64), lambda i: (i, 0)),
        out_shape=jax.ShapeDtypeStruct((n, 64), jnp.float32),
    )(q0, q1, b2)


def kernel(x, adj_t, Wl1, Wr1, att1, b1, Wl2, Wr2, att2, b2):
    loops = jnp.arange(_N, dtype=jnp.int32)
    padi = jnp.full((_EP - _E - _N,), _N, dtype=jnp.int32)
    src = jnp.concatenate([adj_t[0].astype(jnp.int32), loops, padi])
    dst = jnp.concatenate([adj_t[1].astype(jnp.int32), loops, padi])

    gidx1 = jnp.concatenate(
        [src.reshape(-1, _K1), dst.reshape(-1, _K1) + _NP], axis=1)
    dstb1 = dst.reshape(-1, _K1)
    gidx2 = jnp.concatenate(
        [src.reshape(-1, _K2), dst.reshape(-1, _K2) + _NP], axis=1)
    dstb2 = dst.reshape(-1, _K2)

    xp = jnp.pad(x, ((0, _NP - _N), (0, 0)))
    xl1, xr1 = _tc_mm2(xp, Wl1, Wr1)
    t1 = jnp.concatenate([xl1, xr1])

    att1f = att1.reshape(128)
    parts1 = _sc_layer1(t1, gidx1, dstb1, att1f)

    Wl2p = jnp.pad(Wl2, ((0, 0), (0, 8)))
    Wr2p = jnp.pad(Wr2, ((0, 0), (0, 8)))
    hl2, hr2 = _tc_combine1_mm(parts1[0], parts1[1], b1.reshape(1, 128),
                               Wl2p, Wr2p)
    t2 = jnp.concatenate([hl2, hr2])

    att2f = jnp.pad(att2.reshape(40), (0, 8))
    parts2 = _sc_layer2(t2, gidx2, dstb2, att2f)

    b2p = jnp.pad(b2, (0, 24)).reshape(1, 64)
    outp = _tc_combine2(parts2[0], parts2[1], b2p)
    return outp[:_N, :40]
